# Initial kernel scaffold; baseline (speedup 1.0000x reference)
import functools

import jax
import jax.numpy as jnp
from jax import lax
from jax.experimental import pallas as pl
from jax.experimental.pallas import tpu as pltpu, tpu_sc as plsc

N_ATOMS = 10000
E = 160000
F = 128
ND = 20
NDF = ND * F
HARD_CUTOFF = 5.5
CUSP_REG = 1e-06

WA = 80
NWIN = N_ATOMS // WA
NW = 32
WPW = -(-NWIN // NW)
G = 16

BN = 500
BE = 2000
NEB = E // BE


def _proj_body(x_ref, w2d_ref, ws_ref, b_ref, proj_ref, self_ref):
    x = x_ref[...]
    proj_ref[...] = jnp.dot(x, w2d_ref[...], preferred_element_type=jnp.float32)
    self_ref[...] = (
        jnp.dot(x, ws_ref[...], preferred_element_type=jnp.float32) + b_ref[...]
    )


def _tc_proj(feat, w2d, ws_t, b_row):
    return pl.pallas_call(
        _proj_body,
        grid=(N_ATOMS // BN,),
        in_specs=[
            pl.BlockSpec((BN, F), lambda i: (i, 0)),
            pl.BlockSpec((F, NDF), lambda i: (0, 0)),
            pl.BlockSpec((F, F), lambda i: (0, 0)),
            pl.BlockSpec((1, F), lambda i: (0, 0)),
        ],
        out_specs=[
            pl.BlockSpec((BN, NDF), lambda i: (i, 0)),
            pl.BlockSpec((BN, F), lambda i: (i, 0)),
        ],
        out_shape=[
            jax.ShapeDtypeStruct((N_ATOMS, NDF), jnp.float32),
            jax.ShapeDtypeStruct((N_ATOMS, F), jnp.float32),
        ],
    )(feat, w2d, ws_t, b_row)


def _prep_body(dist_ref, cx_ref, cy_ref, cz_ref, pf_ref, mu_ref, sig_ref,
               ed_ref, cnt_ref):
    i = pl.program_id(0)
    dist = dist_ref[...].reshape(BE, 1)
    rinv = 1.0 / dist
    mu = mu_ref[...]
    sig = sig_ref[...]
    z = (rinv - mu) / sig
    cut = jnp.where(
        dist < HARD_CUTOFF,
        jnp.cos((0.5 * jnp.pi / HARD_CUTOFF) * dist) ** 2,
        0.0,
    )
    sense = jnp.exp(-0.5 * z * z) * cut
    ux = cx_ref[...].reshape(BE, 1) * rinv
    uy = cy_ref[...].reshape(BE, 1) * rinv
    uz = cz_ref[...].reshape(BE, 1) * rinv
    col = lax.broadcasted_iota(jnp.int32, (1, 32), 1)
    out = jnp.where(col < ND, sense, 0.0)
    out = jnp.where(col == ND, ux, out)
    out = jnp.where(col == ND + 1, uy, out)
    out = jnp.where(col == ND + 2, uz, out)
    ed_ref[...] = out

    win = pf_ref[...].reshape(BE, 1) // WA
    wiota = lax.broadcasted_iota(jnp.int32, (1, 256), 1)
    onehot = (win == wiota).astype(jnp.int32)
    csum = jnp.sum(onehot, axis=0, keepdims=True)

    @pl.when(i == 0)
    def _():
        cnt_ref[...] = jnp.zeros_like(cnt_ref)

    cnt_ref[...] += csum


def _tc_prep(dist2, cx2, cy2, cz2, pf2, mu_p, sig_p):
    return pl.pallas_call(
        _prep_body,
        grid=(NEB,),
        in_specs=[
            pl.BlockSpec((1, BE), lambda i: (i, 0)),
            pl.BlockSpec((1, BE), lambda i: (i, 0)),
            pl.BlockSpec((1, BE), lambda i: (i, 0)),
            pl.BlockSpec((1, BE), lambda i: (i, 0)),
            pl.BlockSpec((1, BE), lambda i: (i, 0)),
            pl.BlockSpec((1, 32), lambda i: (0, 0)),
            pl.BlockSpec((1, 32), lambda i: (0, 0)),
        ],
        out_specs=[
            pl.BlockSpec((BE, 32), lambda i: (i, 0)),
            pl.BlockSpec((1, 256), lambda i: (0, 0)),
        ],
        out_shape=[
            jax.ShapeDtypeStruct((E, 32), jnp.float32),
            jax.ShapeDtypeStruct((1, 256), jnp.int32),
        ],
    )(dist2, cx2, cy2, cz2, pf2, mu_p, sig_p)


def _extract_i32(ref, idx):
    base = (idx // G) * G
    vec = ref[pl.ds(base, G)]
    lane = idx - base
    m = lax.iota(jnp.int32, G) == lane
    return jnp.sum(jnp.where(m, vec, 0))


def _sc_envsum_body(proj_hbm, ps_hbm, pf_hbm, ed_hbm, offs_hbm, out_hbm,
                    offs_v, idx_v, pf_v, ed_v, rows_v, buf, sem):
    wid = lax.axis_index("c") * 16 + lax.axis_index("s")
    pltpu.sync_copy(offs_hbm, offs_v)

    def do_window(win):
        wbase = win * WA
        lo = _extract_i32(offs_v, win)
        hi = _extract_i32(offs_v, win + 1)

        def zrow(r, _):
            for gg in range(32):
                buf[r, pl.ds(gg * G, G)] = jnp.zeros((G,), jnp.float32)
            return 0

        lax.fori_loop(0, WA, zrow, 0)

        clo = (lo // G) * G
        nch = (hi - clo + (G - 1)) // G

        def chunk(c, _):
            cstart = clo + c * G
            pltpu.sync_copy(ps_hbm.at[pl.ds(cstart, G)], idx_v)
            cp = pltpu.async_copy(proj_hbm.at[idx_v], rows_v, sem)
            pltpu.sync_copy(pf_hbm.at[pl.ds(cstart, G)], pf_v)
            pltpu.sync_copy(ed_hbm.at[pl.ds(cstart, G)], ed_v)
            cp.wait()

            def edge(j, _):
                e = cstart + j
                valid = (e >= lo) & (e < hi)
                scale = jnp.where(valid, jnp.float32(1.0), jnp.float32(0.0))
                pfj = _extract_i32(pf_v, j)
                row = pfj - wbase
                row = lax.max(0, lax.min(row, WA - 1))
                sv0 = ed_v[j, pl.ds(0, G)]
                sv1 = ed_v[j, pl.ds(G, G)]
                q = [jnp.zeros((G,), jnp.float32) for _ in range(8)]
                for d in range(ND):
                    coef = (sv0[d] if d < G else sv1[d - G]) * scale
                    for gg in range(8):
                        q[gg] = q[gg] + coef * rows_v[j, pl.ds(d * F + gg * G, G)]
                ux = sv1[4]
                uy = sv1[5]
                uz = sv1[6]
                for k, uk in enumerate((None, ux, uy, uz)):
                    for gg in range(8):
                        col = k * F + gg * G
                        val = q[gg] if uk is None else uk * q[gg]
                        buf[row, pl.ds(col, G)] = buf[row, pl.ds(col, G)] + val
                return 0

            lax.fori_loop(0, G, edge, 0)
            return 0

        lax.fori_loop(0, nch, chunk, 0)
        pltpu.sync_copy(buf, out_hbm.at[pl.ds(wbase, WA)])

    def witer(i, _):
        win = wid + NW * i

        @pl.when(win < NWIN)
        def _():
            do_window(win)

        return 0

    lax.fori_loop(0, WPW, witer, 0)


def _sc_envsum(proj, ps, pf, edata, offs):
    kfn = pl.kernel(
        _sc_envsum_body,
        out_type=jax.ShapeDtypeStruct((N_ATOMS, 4 * F), jnp.float32),
        mesh=plsc.VectorSubcoreMesh(core_axis_name="c", subcore_axis_name="s"),
        scratch_types=[
            pltpu.VMEM((128,), jnp.int32),
            pltpu.VMEM((G,), jnp.int32),
            pltpu.VMEM((G,), jnp.int32),
            pltpu.VMEM((G, 32), jnp.float32),
            pltpu.VMEM((G, NDF), jnp.float32),
            pltpu.VMEM((WA, 4 * F), jnp.float32),
            pltpu.SemaphoreType.DMA,
        ],
    )
    return kfn(proj, ps, pf, edata, offs)


def _combine_body(o4_ref, self_ref, vs_ref, out_ref):
    a = o4_ref[...]
    s0 = a[:, 0:F]
    v1 = a[:, F:2 * F]
    v2 = a[:, 2 * F:3 * F]
    v3 = a[:, 3 * F:4 * F]
    fv = jnp.sqrt(v1 * v1 + v2 * v2 + v3 * v3 + CUSP_REG) * vs_ref[...]
    out_ref[...] = s0 + fv + self_ref[...]


def _tc_combine(out4, self_out, vs_row):
    return pl.pallas_call(
        _combine_body,
        grid=(N_ATOMS // BN,),
        in_specs=[
            pl.BlockSpec((BN, 4 * F), lambda i: (i, 0)),
            pl.BlockSpec((BN, F), lambda i: (i, 0)),
            pl.BlockSpec((1, F), lambda i: (0, 0)),
        ],
        out_specs=pl.BlockSpec((BN, F), lambda i: (i, 0)),
        out_shape=jax.ShapeDtypeStruct((N_ATOMS, F), jnp.float32),
    )(out4, self_out, vs_row)


def kernel(in_features, pair_first, pair_second, dist_pairs, coord_pairs,
           int_weights, self_W, self_b, vecscales, mu, sigma):
    w2d = jnp.transpose(int_weights, (2, 0, 1)).reshape(F, NDF)
    ws_t = self_W.T
    b_row = self_b.reshape(1, F)
    vs_row = vecscales.reshape(1, F)
    mu_p = jnp.concatenate([mu, jnp.ones((12,), jnp.float32)]).reshape(1, 32)
    sig_p = jnp.concatenate([sigma, jnp.ones((12,), jnp.float32)]).reshape(1, 32)

    dist2 = dist_pairs.reshape(NEB, BE)
    cpt = coord_pairs.T
    cx2 = cpt[0].reshape(NEB, BE)
    cy2 = cpt[1].reshape(NEB, BE)
    cz2 = cpt[2].reshape(NEB, BE)
    pf2 = pair_first.reshape(NEB, BE)

    proj, self_out = _tc_proj(in_features, w2d, ws_t, b_row)
    edata, counts = _tc_prep(dist2, cx2, cy2, cz2, pf2, mu_p, sig_p)

    offs = jnp.concatenate([
        jnp.zeros((1,), jnp.int32),
        jnp.cumsum(counts[0, :NWIN], dtype=jnp.int32),
        jnp.full((128 - NWIN - 1,), E, jnp.int32),
    ])

    out4 = _sc_envsum(proj, pair_second, pair_first, edata, offs)
    return _tc_combine(out4, self_out, vs_row)


# speedup vs baseline: 26.3429x; 26.3429x over previous
import functools

import jax
import jax.numpy as jnp
from jax import lax
from jax.experimental import pallas as pl
from jax.experimental.pallas import tpu as pltpu, tpu_sc as plsc

N_ATOMS = 10000
E = 160000
F = 128
ND = 20
NDF = ND * F
HARD_CUTOFF = 5.5
CUSP_REG = 1e-06

WA = 80
NWIN = N_ATOMS // WA
NW = 32
WPW = -(-NWIN // NW)
G = 16

BN = 400
BE = 2000
NEB = E // BE


def _proj_body(x_ref, w2d_ref, ws_ref, b_ref, proj_ref, self_ref):
    x = x_ref[...]
    proj_ref[...] = jnp.dot(x, w2d_ref[...], preferred_element_type=jnp.float32)
    self_ref[...] = (
        jnp.dot(x, ws_ref[...], preferred_element_type=jnp.float32) + b_ref[...]
    )


def _tc_proj(feat, w2d, ws_t, b_row):
    return pl.pallas_call(
        _proj_body,
        grid=(N_ATOMS // BN,),
        in_specs=[
            pl.BlockSpec((BN, F), lambda i: (i, 0)),
            pl.BlockSpec((F, NDF), lambda i: (0, 0)),
            pl.BlockSpec((F, F), lambda i: (0, 0)),
            pl.BlockSpec((1, F), lambda i: (0, 0)),
        ],
        out_specs=[
            pl.BlockSpec((BN, NDF), lambda i: (i, 0)),
            pl.BlockSpec((BN, F), lambda i: (i, 0)),
        ],
        out_shape=[
            jax.ShapeDtypeStruct((N_ATOMS, NDF), jnp.float32),
            jax.ShapeDtypeStruct((N_ATOMS, F), jnp.float32),
        ],
    )(feat, w2d, ws_t, b_row)


def _prep_body(dist_ref, cx_ref, cy_ref, cz_ref, pf_ref, mu_ref, sig_ref,
               ed_ref, cnt_ref):
    i = pl.program_id(0)
    dist = dist_ref[...].reshape(BE, 1)
    rinv = 1.0 / dist
    mu = mu_ref[...]
    sig = sig_ref[...]
    z = (rinv - mu) / sig
    cut = jnp.where(
        dist < HARD_CUTOFF,
        jnp.cos((0.5 * jnp.pi / HARD_CUTOFF) * dist) ** 2,
        0.0,
    )
    sense = jnp.exp(-0.5 * z * z) * cut
    ux = cx_ref[...].reshape(BE, 1) * rinv
    uy = cy_ref[...].reshape(BE, 1) * rinv
    uz = cz_ref[...].reshape(BE, 1) * rinv
    col = lax.broadcasted_iota(jnp.int32, (1, 32), 1)
    out = jnp.where(col < ND, sense, 0.0)
    out = jnp.where(col == ND, ux, out)
    out = jnp.where(col == ND + 1, uy, out)
    out = jnp.where(col == ND + 2, uz, out)
    ed_ref[...] = out

    win = pf_ref[...].reshape(BE, 1) // WA
    wiota = lax.broadcasted_iota(jnp.int32, (1, 256), 1)
    onehot = (win == wiota).astype(jnp.int32)
    csum = jnp.sum(onehot, axis=0, keepdims=True)

    @pl.when(i == 0)
    def _():
        cnt_ref[...] = jnp.zeros_like(cnt_ref)

    cnt_ref[...] += csum


def _tc_prep(dist2, cx2, cy2, cz2, pf2, mu_p, sig_p):
    return pl.pallas_call(
        _prep_body,
        grid=(NEB,),
        in_specs=[
            pl.BlockSpec((1, 1, BE), lambda i: (i, 0, 0)),
            pl.BlockSpec((1, 1, BE), lambda i: (i, 0, 0)),
            pl.BlockSpec((1, 1, BE), lambda i: (i, 0, 0)),
            pl.BlockSpec((1, 1, BE), lambda i: (i, 0, 0)),
            pl.BlockSpec((1, 1, BE), lambda i: (i, 0, 0)),
            pl.BlockSpec((1, 32), lambda i: (0, 0)),
            pl.BlockSpec((1, 32), lambda i: (0, 0)),
        ],
        out_specs=[
            pl.BlockSpec((BE, 32), lambda i: (i, 0)),
            pl.BlockSpec((1, 256), lambda i: (0, 0)),
        ],
        out_shape=[
            jax.ShapeDtypeStruct((E, 32), jnp.float32),
            jax.ShapeDtypeStruct((1, 256), jnp.int32),
        ],
    )(dist2, cx2, cy2, cz2, pf2, mu_p, sig_p)


def _extract_i32(ref, idx):
    return ref[pl.ds(idx, G)][0]


def _sc_envsum_body(proj_hbm, ps_hbm, pf_hbm, ed_hbm, offs_hbm, out_hbm,
                    offs_v, idx_v, pf_v, ed_v, rows_v, buf, sem):
    wid = lax.axis_index("c") * 16 + lax.axis_index("s")
    pltpu.sync_copy(offs_hbm, offs_v)

    def do_window(win):
        wbase = win * WA
        lo = _extract_i32(offs_v, win)
        hi = _extract_i32(offs_v, win + 1)

        def zrow(r, _):
            for gg in range(32):
                buf[r, pl.ds(gg * G, G)] = jnp.zeros((G,), jnp.float32)
            return 0

        lax.fori_loop(0, WA, zrow, 0)

        clo = (lo // G) * G
        nch = (hi - clo + (G - 1)) // G

        def chunk(c, _):
            cstart = clo + c * G
            pltpu.sync_copy(ps_hbm.at[pl.ds(cstart, G)], idx_v)
            cp = pltpu.async_copy(proj_hbm.at[idx_v], rows_v, sem)
            pltpu.sync_copy(pf_hbm.at[pl.ds(cstart, G)], pf_v.at[pl.ds(0, G)])
            pltpu.sync_copy(ed_hbm.at[pl.ds(cstart, G)], ed_v)
            cp.wait()

            def edge(j, _):
                e = cstart + j
                valid = (e >= lo) & (e < hi)
                scale = jnp.where(valid, jnp.float32(1.0), jnp.float32(0.0))
                pfj = _extract_i32(pf_v, j)
                row = pfj - wbase
                row = lax.max(0, lax.min(row, WA - 1))
                sv0 = ed_v[j, pl.ds(0, G)]
                sv1 = ed_v[j, pl.ds(G, G)]
                q = [jnp.zeros((G,), jnp.float32) for _ in range(8)]
                for d in range(ND):
                    coef = (sv0[d] if d < G else sv1[d - G]) * scale
                    for gg in range(8):
                        q[gg] = q[gg] + coef * rows_v[j, pl.ds(d * F + gg * G, G)]
                ux = sv1[4]
                uy = sv1[5]
                uz = sv1[6]
                for k, uk in enumerate((None, ux, uy, uz)):
                    for gg in range(8):
                        col = k * F + gg * G
                        val = q[gg] if uk is None else uk * q[gg]
                        buf[row, pl.ds(col, G)] = buf[row, pl.ds(col, G)] + val
                return 0

            lax.fori_loop(0, G, edge, 0)
            return 0

        lax.fori_loop(0, nch, chunk, 0)
        pltpu.sync_copy(buf, out_hbm.at[pl.ds(wbase, WA)])

    def witer(i, _):
        win = wid + NW * i

        @pl.when(win < NWIN)
        def _():
            do_window(win)

        return 0

    lax.fori_loop(0, WPW, witer, 0)


def _sc_envsum(proj, ps, pf, edata, offs):
    kfn = pl.kernel(
        _sc_envsum_body,
        out_type=jax.ShapeDtypeStruct((N_ATOMS, 4 * F), jnp.float32),
        mesh=plsc.VectorSubcoreMesh(core_axis_name="c", subcore_axis_name="s"),
        scratch_types=[
            pltpu.VMEM((160,), jnp.int32),
            pltpu.VMEM((G,), jnp.int32),
            pltpu.VMEM((2 * G,), jnp.int32),
            pltpu.VMEM((G, 32), jnp.float32),
            pltpu.VMEM((G, NDF), jnp.float32),
            pltpu.VMEM((WA, 4 * F), jnp.float32),
            pltpu.SemaphoreType.DMA,
        ],
    )
    return kfn(proj, ps, pf, edata, offs)


def _combine_body(o4_ref, self_ref, vs_ref, out_ref):
    a = o4_ref[...]
    s0 = a[:, 0:F]
    v1 = a[:, F:2 * F]
    v2 = a[:, 2 * F:3 * F]
    v3 = a[:, 3 * F:4 * F]
    fv = jnp.sqrt(v1 * v1 + v2 * v2 + v3 * v3 + CUSP_REG) * vs_ref[...]
    out_ref[...] = s0 + fv + self_ref[...]


def _tc_combine(out4, self_out, vs_row):
    return pl.pallas_call(
        _combine_body,
        grid=(N_ATOMS // BN,),
        in_specs=[
            pl.BlockSpec((BN, 4 * F), lambda i: (i, 0)),
            pl.BlockSpec((BN, F), lambda i: (i, 0)),
            pl.BlockSpec((1, F), lambda i: (0, 0)),
        ],
        out_specs=pl.BlockSpec((BN, F), lambda i: (i, 0)),
        out_shape=jax.ShapeDtypeStruct((N_ATOMS, F), jnp.float32),
    )(out4, self_out, vs_row)


def kernel(in_features, pair_first, pair_second, dist_pairs, coord_pairs,
           int_weights, self_W, self_b, vecscales, mu, sigma):
    w2d = jnp.transpose(int_weights, (2, 0, 1)).reshape(F, NDF)
    ws_t = self_W.T
    b_row = self_b.reshape(1, F)
    vs_row = vecscales.reshape(1, F)
    mu_p = jnp.concatenate([mu, jnp.ones((12,), jnp.float32)]).reshape(1, 32)
    sig_p = jnp.concatenate([sigma, jnp.ones((12,), jnp.float32)]).reshape(1, 32)

    dist2 = dist_pairs.reshape(NEB, 1, BE)
    cpt = coord_pairs.T
    cx2 = cpt[0].reshape(NEB, 1, BE)
    cy2 = cpt[1].reshape(NEB, 1, BE)
    cz2 = cpt[2].reshape(NEB, 1, BE)
    pf2 = pair_first.reshape(NEB, 1, BE)

    proj, self_out = _tc_proj(in_features, w2d, ws_t, b_row)
    edata, counts = _tc_prep(dist2, cx2, cy2, cz2, pf2, mu_p, sig_p)

    offs = jnp.concatenate([
        jnp.zeros((1,), jnp.int32),
        jnp.cumsum(counts[0, :NWIN], dtype=jnp.int32),
        jnp.full((160 - NWIN - 1,), E, jnp.int32),
    ])

    out4 = _sc_envsum(proj, pair_second, pair_first, edata, offs)
    return _tc_combine(out4, self_out, vs_row)
